# Initial kernel scaffold; baseline (speedup 1.0000x reference)
import jax
import jax.numpy as jnp
from jax.experimental import pallas as pl


def kernel(fake, tar, sh, sw):
    raise NotImplementedError("write your pallas kernel here")


# speedup vs baseline: 1.3419x; 1.3419x over previous
import math

import jax
import jax.numpy as jnp
from jax.experimental import pallas as pl
from jax.experimental.pallas import tpu as pltpu

_W_ORI = 1285
_H_ORI = 438
_FARO_V = 123.5
_FARO_H = 360.0
_CROP = 384

_N = 128 * 128
_TA = 256
_TB = 2048
_NI = _N // _TA
_NJ = _N // _TB
_K = 8


def _directions(h, w, sh, sw):
    fv = _FARO_V * _CROP / _H_ORI
    fh = _FARO_H * _CROP / _W_ORI
    cw_rad = sw / _W_ORI * _FARO_H
    ch_rad = sh / _H_ORI * _FARO_V
    p, q = jnp.meshgrid(jnp.arange(h), jnp.arange(w), indexing="ij")
    points_hw = jnp.stack([p, q], axis=-1).reshape(-1, 2).astype(jnp.float32)
    yaw = (-fh * points_hw[:, 1] / w + cw_rad) * (math.pi / 180.0)
    pitch = (-fv * points_hw[:, 0] / h + ch_rad) * (math.pi / 180.0)
    ux = jnp.sin(yaw) * jnp.sin(pitch)
    uy = jnp.cos(yaw) * jnp.sin(pitch)
    uz = jnp.cos(pitch)
    return ux, uy, uz


def _chamfer_body(a_ref, p2_ref, b_ref, q2_ref, out_ref, dist2_ref, sum1_ref):
    i = pl.program_id(0)

    @pl.when(i == 0)
    def _init():
        dist2_ref[...] = jnp.full((1, _N), jnp.inf, jnp.float32)
        sum1_ref[0] = 0.0

    a_blk = a_ref[...]
    p2_blk = p2_ref[...]

    rowmin = jnp.full((_TA, 1), jnp.inf, jnp.float32)
    for j in range(_NJ):
        cols = pl.ds(j * _TB, _TB)
        ab = jax.lax.dot_general(
            a_blk, b_ref[:, cols],
            (((1,), (0,)), ((), ())),
            preferred_element_type=jnp.float32)
        d = p2_blk - 2.0 * ab + q2_ref[:, cols]
        rowmin = jnp.minimum(rowmin, jnp.min(d, axis=1, keepdims=True))
        dist2_ref[:, cols] = jnp.minimum(
            dist2_ref[:, cols], jnp.min(d, axis=0, keepdims=True))
    sum1_ref[0] += jnp.sum(rowmin)

    @pl.when(i == _NI - 1)
    def _fin():
        out_ref[0] = (sum1_ref[0] + jnp.sum(dist2_ref[...])) / float(_N)


def kernel(fake, tar, sh, sw):
    b, _, h, w = fake.shape
    ux, uy, uz = _directions(h, w, sh, sw)
    dp = tar[b - 1, 0].reshape(-1)
    dq = fake[b - 1, 0].reshape(-1)

    px, py, pz = dp * ux, dp * uy, dp * uz
    qx, qy, qz = dq * ux, dq * uy, dq * uz
    zero = jnp.zeros((_N,), jnp.float32)
    a_mat = jnp.stack([px, py, pz, zero, zero, zero, zero, zero],
                      axis=1).astype(jnp.bfloat16)
    b_mat = jnp.stack([qx, qy, qz, zero, zero, zero, zero, zero],
                      axis=0).astype(jnp.bfloat16)
    p2 = (px * px + py * py + pz * pz).reshape(_N, 1)
    q2 = (qx * qx + qy * qy + qz * qz).reshape(1, _N)

    out = pl.pallas_call(
        _chamfer_body,
        grid=(_NI,),
        in_specs=[
            pl.BlockSpec((_TA, _K), lambda i: (i, 0)),
            pl.BlockSpec((_TA, 1), lambda i: (i, 0)),
            pl.BlockSpec((_K, _N), lambda i: (0, 0)),
            pl.BlockSpec((1, _N), lambda i: (0, 0)),
        ],
        out_specs=pl.BlockSpec(memory_space=pltpu.SMEM),
        out_shape=jax.ShapeDtypeStruct((1,), jnp.float32),
        scratch_shapes=[
            pltpu.VMEM((1, _N), jnp.float32),
            pltpu.SMEM((1,), jnp.float32),
        ],
    )(a_mat, p2, b_mat, q2)
    return out[0]


# speedup vs baseline: 2.8959x; 2.1581x over previous
import math

import jax
import jax.numpy as jnp
from jax.experimental import pallas as pl
from jax.experimental.pallas import tpu as pltpu

_W_ORI = 1285
_H_ORI = 438
_FARO_V = 123.5
_FARO_H = 360.0
_CROP = 384

_N = 128 * 128
_TA = 256
_TB = 2048
_NI = _N // _TA
_NJ = _N // _TB
_K = 8


def _directions(h, w, sh, sw):
    fv = _FARO_V * _CROP / _H_ORI
    fh = _FARO_H * _CROP / _W_ORI
    cw_rad = sw / _W_ORI * _FARO_H
    ch_rad = sh / _H_ORI * _FARO_V
    p, q = jnp.meshgrid(jnp.arange(h), jnp.arange(w), indexing="ij")
    points_hw = jnp.stack([p, q], axis=-1).reshape(-1, 2).astype(jnp.float32)
    yaw = (-fh * points_hw[:, 1] / w + cw_rad) * (math.pi / 180.0)
    pitch = (-fv * points_hw[:, 0] / h + ch_rad) * (math.pi / 180.0)
    ux = jnp.sin(yaw) * jnp.sin(pitch)
    uy = jnp.cos(yaw) * jnp.sin(pitch)
    uz = jnp.cos(pitch)
    return ux, uy, uz


def _chamfer_body(a_ref, b_ref, out_ref, dist2_ref, sum1_ref):
    i = pl.program_id(0)

    @pl.when(i == 0)
    def _init():
        dist2_ref[...] = jnp.full((1, _N), jnp.inf, jnp.float32)
        sum1_ref[0] = 0.0

    a_blk = a_ref[...]

    rowmin = jnp.full((_TA, 1), jnp.inf, jnp.float32)
    for j in range(_NJ):
        cols = pl.ds(j * _TB, _TB)
        d = jax.lax.dot_general(
            a_blk, b_ref[:, cols],
            (((1,), (0,)), ((), ())),
            preferred_element_type=jnp.float32)
        rowmin = jnp.minimum(rowmin, jnp.min(d, axis=1, keepdims=True))
        dist2_ref[:, cols] = jnp.minimum(
            dist2_ref[:, cols], jnp.min(d, axis=0, keepdims=True))
    sum1_ref[0] += jnp.sum(rowmin)

    @pl.when(i == _NI - 1)
    def _fin():
        out_ref[0] = (sum1_ref[0] + jnp.sum(dist2_ref[...])) / float(_N)


def kernel(fake, tar, sh, sw):
    b, _, h, w = fake.shape
    ux, uy, uz = _directions(h, w, sh, sw)
    dp = tar[b - 1, 0].reshape(-1)
    dq = fake[b - 1, 0].reshape(-1)

    px, py, pz = dp * ux, dp * uy, dp * uz
    qx, qy, qz = dq * ux, dq * uy, dq * uz
    p2 = px * px + py * py + pz * pz
    q2 = qx * qx + qy * qy + qz * qz
    p2_hi = p2.astype(jnp.bfloat16)
    p2_lo = (p2 - p2_hi.astype(jnp.float32)).astype(jnp.bfloat16)
    q2_hi = q2.astype(jnp.bfloat16)
    q2_lo = (q2 - q2_hi.astype(jnp.float32)).astype(jnp.bfloat16)
    one = jnp.ones((_N,), jnp.float32)
    zero = jnp.zeros((_N,), jnp.float32)
    m2 = jnp.float32(-2.0)
    a_mat = jnp.stack(
        [(m2 * px).astype(jnp.bfloat16), (m2 * py).astype(jnp.bfloat16),
         (m2 * pz).astype(jnp.bfloat16), p2_hi, p2_lo,
         one.astype(jnp.bfloat16), one.astype(jnp.bfloat16),
         zero.astype(jnp.bfloat16)], axis=1)
    b_mat = jnp.stack(
        [qx.astype(jnp.bfloat16),
         qy.astype(jnp.bfloat16), qz.astype(jnp.bfloat16),
         one.astype(jnp.bfloat16), one.astype(jnp.bfloat16),
         q2_hi, q2_lo, zero.astype(jnp.bfloat16)], axis=0)

    out = pl.pallas_call(
        _chamfer_body,
        grid=(_NI,),
        in_specs=[
            pl.BlockSpec((_TA, _K), lambda i: (i, 0)),
            pl.BlockSpec((_K, _N), lambda i: (0, 0)),
        ],
        out_specs=pl.BlockSpec(memory_space=pltpu.SMEM),
        out_shape=jax.ShapeDtypeStruct((1,), jnp.float32),
        scratch_shapes=[
            pltpu.VMEM((1, _N), jnp.float32),
            pltpu.SMEM((1,), jnp.float32),
        ],
    )(a_mat, b_mat)
    return out[0]
